# baseline (speedup 1.0000x reference)
import math
import functools

import jax
import jax.numpy as jnp
from jax import lax
from jax.experimental import pallas as pl
from jax.experimental.pallas import tpu as pltpu

_N = 10000
_E = 160000
_B = 64
_CTX = 512
_PED = 240
_HID = 128
_IND = 880
_OUT = 1024
_BN = 1000
_BNSCALE = 1.0 / math.sqrt(1.0 + 1e-5)


def _sinpe(positions, D):
    div_term = jnp.exp(jnp.arange(0, D // 2, dtype=jnp.float32) * -(jnp.log(10000.0) / (D // 2)))
    cd = D // 3
    parts = []
    for i in range(3):
        pos = positions[:, i][:, None]
        s = pos * div_term[: cd // 2]
        parts.append(jnp.concatenate([jnp.sin(s), jnp.cos(s)], axis=-1))
    return jnp.concatenate(parts, axis=1)


def _layer_update_body(xc_ref, aggr_ref, h_ref, wn_ref, bn_ref, g_ref, bt_ref, hn_ref):
    xa = xc_ref[...] + aggr_ref[...]
    out = jnp.dot(xa, wn_ref[...], preferred_element_type=jnp.float32)
    out = (out + bn_ref[...]) * (_BNSCALE * g_ref[...]) + bt_ref[...]
    hn_ref[...] = h_ref[...] + 0.5 * out * (1.0 + lax.erf(out / math.sqrt(2.0)))


def _layer_update(xc, aggr, h, wn, bn, g, bt):
    grid = (_N // _BN,)
    return pl.pallas_call(
        _layer_update_body,
        grid=grid,
        in_specs=[
            pl.BlockSpec((_BN, _IND), lambda i: (i, 0)),
            pl.BlockSpec((_BN, _IND), lambda i: (i, 0)),
            pl.BlockSpec((_BN, _HID), lambda i: (i, 0)),
            pl.BlockSpec((_IND, _HID), lambda i: (0, 0)),
            pl.BlockSpec((1, _HID), lambda i: (0, 0)),
            pl.BlockSpec((1, _HID), lambda i: (0, 0)),
            pl.BlockSpec((1, _HID), lambda i: (0, 0)),
        ],
        out_specs=pl.BlockSpec((_BN, _HID), lambda i: (i, 0)),
        out_shape=jax.ShapeDtypeStruct((_N, _HID), jnp.float32),
    )(xc, aggr, h, wn, bn.reshape(1, -1), g.reshape(1, -1), bt.reshape(1, -1))


def _pool_body(h_ref, oh_ref, acc_ref, m_ref):
    i = pl.program_id(0)

    @pl.when(i == 0)
    def _():
        acc_ref[...] = jnp.zeros_like(acc_ref)

    acc_ref[...] += jnp.dot(oh_ref[...].T, h_ref[...], preferred_element_type=jnp.float32)

    @pl.when(i == pl.num_programs(0) - 1)
    def _():
        m_ref[...] = acc_ref[...]


def _pool_h(h, onehot_b):
    grid = (_N // _BN,)
    return pl.pallas_call(
        _pool_body,
        grid=grid,
        in_specs=[
            pl.BlockSpec((_BN, _HID), lambda i: (i, 0)),
            pl.BlockSpec((_BN, _B), lambda i: (i, 0)),
        ],
        out_specs=pl.BlockSpec((_B, _HID), lambda i: (0, 0)),
        out_shape=jax.ShapeDtypeStruct((_B, _HID), jnp.float32),
        scratch_shapes=[pltpu.VMEM((_B, _HID), jnp.float32)],
    )(h, onehot_b)


def kernel(x, pos, edge_index, edge_attr, batch, context_vector, W0, b0, Wn0, bn0, We0, be0, g0, bt0, Wn1, bn1, We1, be1, g1, bt1, Wn2, bn2, We2, be2, g2, bt2, Wl, bl):
    h = jax.nn.one_hot(x.flatten(), 118, dtype=jnp.float32) @ W0 + b0
    pe = _sinpe(pos, _PED)
    ctxb = context_vector[batch]
    src = edge_index[0]
    dst = edge_index[1]
    onehot_b = jax.nn.one_hot(batch, _B, dtype=jnp.float32)

    layers = [(Wn0, bn0, We0, be0, g0, bt0),
              (Wn1, bn1, We1, be1, g1, bt1),
              (Wn2, bn2, We2, be2, g2, bt2)]
    for (Wn, bn, We, be, g, bt) in layers:
        xc = jnp.concatenate([h, pe, ctxb], axis=1)
        eemb = edge_attr @ We + be
        msg = jax.nn.relu(xc[src] + eemb)
        aggr = jax.ops.segment_sum(msg, dst, num_segments=_N)
        h = _layer_update(xc, aggr, h, Wn, bn, g, bt)

    m = _pool_h(h, onehot_b)
    cnt = jnp.sum(onehot_b, axis=0)
    return m @ Wl + cnt[:, None] * bl


# speedup vs baseline: 1.5627x; 1.5627x over previous
import functools
import math

import jax
import jax.numpy as jnp
from jax import lax
from jax.experimental import pallas as pl
from jax.experimental.pallas import tpu as pltpu
from jax.experimental.pallas import tpu_sc as plsc

_N = 10000
_E = 160000
_B = 64
_CTX = 512
_PED = 240
_HID = 128
_IND = 880
_OUT = 1024

_BN = 1000
_BE = 2000
_NCHUNK = 7
_CW = 128
_INDP = _NCHUNK * _CW
_VPC = _CW // 16
_K = 128
_NBLK = _E // _K
_NSUB = 16
_NPAD = 10240
_NZCH = _NPAD // _K
_NFCH = _N // _K
_BNSCALE = 1.0 / math.sqrt(1.0 + 1e-5)


def _sinpe(positions, D):
    div_term = jnp.exp(jnp.arange(0, D // 2, dtype=jnp.float32) * -(jnp.log(10000.0) / (D // 2)))
    cd = D // 3
    parts = []
    for i in range(3):
        pos = positions[:, i][:, None]
        s = pos * div_term[: cd // 2]
        parts.append(jnp.concatenate([jnp.sin(s), jnp.cos(s)], axis=-1))
    return jnp.concatenate(parts, axis=1)


def _eemb_body(ea_ref, *refs):
    ws = refs[:_NCHUNK]
    bs = refs[_NCHUNK:2 * _NCHUNK]
    outs = refs[2 * _NCHUNK:]
    ea = ea_ref[...]
    for c in range(_NCHUNK):
        outs[c][...] = jnp.dot(ea, ws[c][...], preferred_element_type=jnp.float32) + bs[c][...]


def _eemb_chunks(edge_attr, We, be):
    grid = (_E // _BE,)
    in_specs = [pl.BlockSpec((_BE, 5), lambda i: (i, 0))]
    in_specs += [pl.BlockSpec((5, _CW), lambda i: (0, 0))] * _NCHUNK
    in_specs += [pl.BlockSpec((1, _CW), lambda i: (0, 0))] * _NCHUNK
    args = [edge_attr]
    args += [We[:, c * _CW:(c + 1) * _CW] for c in range(_NCHUNK)]
    args += [be[c * _CW:(c + 1) * _CW].reshape(1, _CW) for c in range(_NCHUNK)]
    return pl.pallas_call(
        _eemb_body,
        grid=grid,
        in_specs=in_specs,
        out_specs=[pl.BlockSpec((_BE, _CW), lambda i: (i, 0))] * _NCHUNK,
        out_shape=[jax.ShapeDtypeStruct((_E, _CW), jnp.float32)] * _NCHUNK,
    )(*args)


def _edge_sc_body(src_h, dst_h, *refs):
    xcs = refs[:_NCHUNK]
    ees = refs[_NCHUNK:2 * _NCHUNK]
    outs = refs[2 * _NCHUNK:3 * _NCHUNK]
    acc, srcv, dstv, gbuf, ebuf, sem = refs[3 * _NCHUNK:]
    c = lax.axis_index("c")
    s = lax.axis_index("s")

    def zero_gbuf():
        def zrow(j, carry):
            for v in range(_VPC):
                gbuf[j, pl.ds(v * 16, 16)] = jnp.zeros((16,), jnp.float32)
            return carry
        lax.fori_loop(0, _K, zrow, 0)

    def process(xcc, eec, outc):
        zero_gbuf()
        for k in range(_NZCH // _NSUB):
            pltpu.sync_copy(gbuf, acc.at[pl.ds((k * _NSUB + s) * _K, _K)])
        plsc.subcore_barrier()

        def eblk(i, carry):
            blk = i * _NSUB + s

            @pl.when(blk < _NBLK)
            def _():
                e0 = blk * _K
                pltpu.sync_copy(src_h.at[pl.ds(e0, _K)], srcv)
                pltpu.sync_copy(dst_h.at[pl.ds(e0, _K)], dstv)
                pltpu.async_copy(xcc.at[srcv], gbuf, sem).wait()
                pltpu.sync_copy(eec.at[pl.ds(e0, _K)], ebuf)

                def crow(j, carry2):
                    for v in range(_VPC):
                        sl = pl.ds(v * 16, 16)
                        gbuf[j, sl] = jnp.maximum(gbuf[j, sl] + ebuf[j, sl], 0.0)
                    return carry2
                lax.fori_loop(0, _K, crow, 0)
                pltpu.sync_copy(gbuf, acc.at[dstv], add=True)
            return carry
        lax.fori_loop(0, (_NBLK + _NSUB - 1) // _NSUB, eblk, 0)
        plsc.subcore_barrier()

        for k in range((_NFCH + _NSUB - 1) // _NSUB):
            idx = k * _NSUB + s

            @pl.when(idx < _NFCH)
            def _():
                off = idx * _K
                pltpu.sync_copy(acc.at[pl.ds(off, _K)], gbuf)
                pltpu.sync_copy(gbuf, outc.at[pl.ds(off, _K)])

        @pl.when(s == 0)
        def _():
            pltpu.sync_copy(acc.at[pl.ds(_NFCH * _K, _N - _NFCH * _K)],
                            gbuf.at[pl.ds(0, _N - _NFCH * _K)])
            pltpu.sync_copy(gbuf.at[pl.ds(0, _N - _NFCH * _K)],
                            outc.at[pl.ds(_NFCH * _K, _N - _NFCH * _K)])
        plsc.subcore_barrier()

    @pl.when(c == 0)
    def _():
        for ch in range(0, _NCHUNK, 2):
            process(xcs[ch], ees[ch], outs[ch])

    @pl.when(c == 1)
    def _():
        for ch in range(1, _NCHUNK, 2):
            process(xcs[ch], ees[ch], outs[ch])


def _edge_pass_sc(src, dst, xc_chunks, ee_chunks):
    mesh = plsc.VectorSubcoreMesh(core_axis_name="c", subcore_axis_name="s")
    fn = pl.kernel(
        _edge_sc_body,
        out_type=[jax.ShapeDtypeStruct((_N, _CW), jnp.float32)] * _NCHUNK,
        mesh=mesh,
        scratch_types=[
            pltpu.VMEM_SHARED((_NPAD, _CW), jnp.float32),
            pltpu.VMEM((_K,), jnp.int32),
            pltpu.VMEM((_K,), jnp.int32),
            pltpu.VMEM((_K, _CW), jnp.float32),
            pltpu.VMEM((_K, _CW), jnp.float32),
            pltpu.SemaphoreType.DMA,
        ],
    )
    return fn(src, dst, *xc_chunks, *ee_chunks)


def _layer_update_body(xc_ref, *refs):
    achunks = refs[:_NCHUNK]
    h_ref, wn_ref, bn_ref, g_ref, bt_ref, hn_ref = refs[_NCHUNK:]
    aggr = jnp.concatenate([a[...] for a in achunks], axis=1)
    xa = xc_ref[...] + aggr
    out = jnp.dot(xa, wn_ref[...], preferred_element_type=jnp.float32)
    out = (out + bn_ref[...]) * (_BNSCALE * g_ref[...]) + bt_ref[...]
    hn_ref[...] = h_ref[...] + 0.5 * out * (1.0 + lax.erf(out / math.sqrt(2.0)))


def _layer_update(xc, aggr_chunks, h, wn, bn, g, bt):
    grid = (_N // _BN,)
    in_specs = [pl.BlockSpec((_BN, _INDP), lambda i: (i, 0))]
    in_specs += [pl.BlockSpec((_BN, _CW), lambda i: (i, 0))] * _NCHUNK
    in_specs += [
        pl.BlockSpec((_BN, _HID), lambda i: (i, 0)),
        pl.BlockSpec((_INDP, _HID), lambda i: (0, 0)),
        pl.BlockSpec((1, _HID), lambda i: (0, 0)),
        pl.BlockSpec((1, _HID), lambda i: (0, 0)),
        pl.BlockSpec((1, _HID), lambda i: (0, 0)),
    ]
    return pl.pallas_call(
        _layer_update_body,
        grid=grid,
        in_specs=in_specs,
        out_specs=pl.BlockSpec((_BN, _HID), lambda i: (i, 0)),
        out_shape=jax.ShapeDtypeStruct((_N, _HID), jnp.float32),
    )(xc, *aggr_chunks, h, wn, bn.reshape(1, -1), g.reshape(1, -1), bt.reshape(1, -1))


def _pool_body(h_ref, oh_ref, wl_ref, bl_ref, out_ref, acc_ref, cnt_ref):
    i = pl.program_id(0)

    @pl.when(i == 0)
    def _():
        acc_ref[...] = jnp.zeros_like(acc_ref)
        cnt_ref[...] = jnp.zeros_like(cnt_ref)

    oh = oh_ref[...]
    acc_ref[...] += jnp.dot(oh.T, h_ref[...], preferred_element_type=jnp.float32)
    cnt_ref[...] += jnp.sum(oh, axis=0, keepdims=True)

    @pl.when(i == pl.num_programs(0) - 1)
    def _():
        m = jnp.dot(acc_ref[...], wl_ref[...], preferred_element_type=jnp.float32)
        out_ref[...] = m + cnt_ref[...].reshape(_B, 1) * bl_ref[...]


def _pool_project(h, onehot_b, wl, bl):
    grid = (_N // _BN,)
    return pl.pallas_call(
        _pool_body,
        grid=grid,
        in_specs=[
            pl.BlockSpec((_BN, _HID), lambda i: (i, 0)),
            pl.BlockSpec((_BN, _B), lambda i: (i, 0)),
            pl.BlockSpec((_HID, _OUT), lambda i: (0, 0)),
            pl.BlockSpec((1, _OUT), lambda i: (0, 0)),
        ],
        out_specs=pl.BlockSpec((_B, _OUT), lambda i: (0, 0)),
        out_shape=jax.ShapeDtypeStruct((_B, _OUT), jnp.float32),
        scratch_shapes=[pltpu.VMEM((_B, _HID), jnp.float32),
                        pltpu.VMEM((1, _B), jnp.float32)],
    )(h, onehot_b, wl, bl.reshape(1, -1))


def kernel(x, pos, edge_index, edge_attr, batch, context_vector, W0, b0, Wn0, bn0, We0, be0, g0, bt0, Wn1, bn1, We1, be1, g1, bt1, Wn2, bn2, We2, be2, g2, bt2, Wl, bl):
    h = jax.nn.one_hot(x.flatten(), 118, dtype=jnp.float32) @ W0 + b0
    pe = _sinpe(pos, _PED)
    ctxb = context_vector[batch]
    src = edge_index[0]
    dst = edge_index[1]
    onehot_b = jax.nn.one_hot(batch, _B, dtype=jnp.float32)

    layers = [(Wn0, bn0, We0, be0, g0, bt0),
              (Wn1, bn1, We1, be1, g1, bt1),
              (Wn2, bn2, We2, be2, g2, bt2)]
    zpad = jnp.zeros((_N, _INDP - _IND), jnp.float32)
    for (Wn, bn, We, be, g, bt) in layers:
        xc = jnp.concatenate([h, pe, ctxb, zpad], axis=1)
        xc_chunks = [xc[:, c * _CW:(c + 1) * _CW] for c in range(_NCHUNK)]
        we_pad = jnp.pad(We, ((0, 0), (0, _INDP - _IND)))
        be_pad = jnp.pad(be, (0, _INDP - _IND))
        wn_pad = jnp.pad(Wn, ((0, _INDP - _IND), (0, 0)))
        ee_chunks = _eemb_chunks(edge_attr, we_pad, be_pad)
        aggr_chunks = _edge_pass_sc(src, dst, xc_chunks, ee_chunks)
        h = _layer_update(xc, aggr_chunks, h, wn_pad, bn, g, bt)

    return _pool_project(h, onehot_b, Wl, bl)
